# baseline (speedup 1.0000x reference)
import functools

import jax
import jax.numpy as jnp
from jax import lax
from jax.experimental import pallas as pl
from jax.experimental.pallas import tpu as pltpu
from jax.experimental.pallas import tpu_sc as plsc

EMB_DIM = 128


@functools.lru_cache(maxsize=None)
def _build(n, pw, E):
    info = plsc.get_sparse_core_info()
    NC, NS, L = info.num_cores, info.num_subcores, info.num_lanes
    NW = NC * NS
    BPW = E // NW
    CH = 128
    NCH = BPW // CH
    HR = BPW // 2
    ZC = EMB_DIM - pw

    mesh = plsc.VectorSubcoreMesh(core_axis_name="c", subcore_axis_name="s")

    def body(tab, src, dst, zsrc, out, src_v, dst_v, idx_v, rows_v, zer_v,
             sem, zsem):
        wid = lax.axis_index("s") * NC + lax.axis_index("c")
        base = wid * BPW
        zcp = pltpu.async_copy(zsrc, zer_v, zsem)
        pltpu.sync_copy(src.at[pl.ds(base, BPW)], src_v)
        pltpu.sync_copy(dst.at[pl.ds(base, BPW)], dst_v)
        for i in range(BPW // L):
            s16 = src_v[pl.ds(i * L, L)]
            d16 = dst_v[pl.ds(i * L, L)]
            idx_v[i // (CH // L), pl.ds((i % (CH // L)) * L, L)] = (
                s16 * n + lax.rem(d16, n))
        cps = [
            pltpu.async_copy(tab.at[idx_v.at[j]],
                             rows_v.at[pl.ds(j * CH, CH)], sem)
            for j in range(NCH)
        ]
        zcp.wait()
        pltpu.sync_copy(zer_v, out.at[pl.ds(base, HR), pl.ds(pw, ZC)])
        pltpu.sync_copy(zer_v, out.at[pl.ds(base + HR, HR), pl.ds(pw, ZC)])
        for cp in cps:
            cp.wait()
        pltpu.sync_copy(rows_v, out.at[pl.ds(base, BPW), pl.ds(0, pw)])

    return pl.kernel(
        body,
        mesh=mesh,
        out_type=jax.ShapeDtypeStruct((E, EMB_DIM), jnp.float32),
        scratch_types=[
            pltpu.VMEM((BPW,), jnp.int32),
            pltpu.VMEM((BPW,), jnp.int32),
            pltpu.VMEM((NCH, CH), jnp.int32),
            pltpu.VMEM((BPW, pw), jnp.float32),
            pltpu.VMEM((HR, ZC), jnp.float32),
            pltpu.SemaphoreType.DMA,
            pltpu.SemaphoreType.DMA,
        ],
        compiler_params=pltpu.CompilerParams(use_tc_tiling_on_sc=False),
    )


@functools.lru_cache(maxsize=None)
def _build_tpose(pe, pw, V, CW):
    def tbody(x_ref, o_ref):
        xt = jnp.transpose(x_ref[...])
        o_ref[...] = jnp.concatenate(
            [xt, jnp.zeros((CW, pw - pe), jnp.float32)], axis=1)

    return pl.pallas_call(
        tbody,
        grid=(V // CW,),
        in_specs=[pl.BlockSpec((pe, CW), lambda i: (0, i))],
        out_specs=pl.BlockSpec((CW, pw), lambda i: (i, 0)),
        out_shape=jax.ShapeDtypeStruct((V, pw), jnp.float32),
    )


def kernel(edge_RWSE, batch, edge_index):
    total_nodes = batch.shape[0]
    n = edge_RWSE.shape[0] // total_nodes
    pe = edge_RWSE.shape[1]
    V = edge_RWSE.shape[0]
    E = edge_index.shape[1]
    pw = ((pe + 7) // 8) * 8
    tab = _build_tpose(pe, pw, V, 512)(edge_RWSE.T)
    ei = edge_index.astype(jnp.int32)
    zsrc = jnp.zeros((E // 64, EMB_DIM - pw), jnp.float32)
    return _build(n, pw, E)(tab, ei[0], ei[1], zsrc)


# speedup vs baseline: 2.0292x; 2.0292x over previous
import functools

import jax
import jax.numpy as jnp
from jax import lax
from jax.experimental import pallas as pl
from jax.experimental.pallas import tpu as pltpu
from jax.experimental.pallas import tpu_sc as plsc

EMB_DIM = 128


@functools.lru_cache(maxsize=None)
def _build(n, pw, E):
    info = plsc.get_sparse_core_info()
    NC, NS, L = info.num_cores, info.num_subcores, info.num_lanes
    NW = NC * NS
    BPW = E // NW
    CH = 128
    NCH = BPW // CH
    HR = BPW // 2
    ZC = EMB_DIM - pw

    mesh = plsc.VectorSubcoreMesh(core_axis_name="c", subcore_axis_name="s")

    def body(tab, src, dst, zsrc, out, src_v, dst_v, idx_v, rows_v, zer_v,
             sem, zsem):
        wid = lax.axis_index("s") * NC + lax.axis_index("c")
        base = wid * BPW
        zcp = pltpu.async_copy(zsrc, zer_v, zsem)
        pltpu.sync_copy(src.at[pl.ds(base, BPW)], src_v)
        pltpu.sync_copy(dst.at[pl.ds(base, BPW)], dst_v)
        for i in range(BPW // L):
            s16 = src_v[pl.ds(i * L, L)]
            d16 = dst_v[pl.ds(i * L, L)]
            idx_v[i // (CH // L), pl.ds((i % (CH // L)) * L, L)] = (
                s16 * n + lax.rem(d16, n))
        cps = [
            pltpu.async_copy(tab.at[idx_v.at[j]],
                             rows_v.at[pl.ds(j * CH, CH)], sem)
            for j in range(NCH)
        ]
        zcp.wait()
        pltpu.sync_copy(zer_v, out.at[pl.ds(base, HR), pl.ds(pw, ZC)])
        pltpu.sync_copy(zer_v, out.at[pl.ds(base + HR, HR), pl.ds(pw, ZC)])
        for cp in cps:
            cp.wait()
        pltpu.sync_copy(rows_v, out.at[pl.ds(base, BPW), pl.ds(0, pw)])

    return pl.kernel(
        body,
        mesh=mesh,
        out_type=jax.ShapeDtypeStruct((E, EMB_DIM), jnp.float32),
        scratch_types=[
            pltpu.VMEM((BPW,), jnp.int32),
            pltpu.VMEM((BPW,), jnp.int32),
            pltpu.VMEM((NCH, CH), jnp.int32),
            pltpu.VMEM((BPW, pw), jnp.float32),
            pltpu.VMEM((HR, ZC), jnp.float32),
            pltpu.SemaphoreType.DMA,
            pltpu.SemaphoreType.DMA,
        ],
        compiler_params=pltpu.CompilerParams(use_tc_tiling_on_sc=False),
    )


@functools.lru_cache(maxsize=None)
def _build_tpose(pe, pw, V, CW):
    def tbody(x_ref, o_ref):
        xt = jnp.transpose(x_ref[...])
        o_ref[...] = jnp.concatenate(
            [xt, jnp.zeros((CW, pw - pe), jnp.float32)], axis=1)

    return pl.pallas_call(
        tbody,
        grid=(V // CW,),
        in_specs=[pl.BlockSpec((pe, CW), lambda i: (0, i))],
        out_specs=pl.BlockSpec((CW, pw), lambda i: (i, 0)),
        out_shape=jax.ShapeDtypeStruct((V, pw), jnp.float32),
    )


def kernel(edge_RWSE, batch, edge_index):
    total_nodes = batch.shape[0]
    n = edge_RWSE.shape[0] // total_nodes
    pe = edge_RWSE.shape[1]
    V = edge_RWSE.shape[0]
    E = edge_index.shape[1]
    pw = ((pe + 7) // 8) * 8
    tab = _build_tpose(pe, pw, V, 4096)(edge_RWSE.T)
    ei = edge_index.astype(jnp.int32)
    zsrc = jnp.zeros((E // 64, EMB_DIM - pw), jnp.float32)
    return _build(n, pw, E)(tab, ei[0], ei[1], zsrc)


# speedup vs baseline: 4.6112x; 2.2724x over previous
import functools

import jax
import jax.numpy as jnp
from jax import lax
from jax.experimental import pallas as pl
from jax.experimental.pallas import tpu as pltpu
from jax.experimental.pallas import tpu_sc as plsc

EMB_DIM = 128


@functools.lru_cache(maxsize=None)
def _build(n, pe, pw, E):
    info = plsc.get_sparse_core_info()
    NC, NS, L = info.num_cores, info.num_subcores, info.num_lanes
    NW = NC * NS
    BPW = E // NW
    CH = 128
    NCH = BPW // CH
    HR = BPW // 2
    ZC = EMB_DIM - pw

    mesh = plsc.VectorSubcoreMesh(core_axis_name="c", subcore_axis_name="s")

    def body(tabT, src, dst, zsrc, out, src_v, dst_v, idx_v, col_v, rows_v,
             zer_v, sem, zsem):
        wid = lax.axis_index("s") * NC + lax.axis_index("c")
        base = wid * BPW
        zcp = pltpu.async_copy(zsrc, zer_v, zsem)
        pltpu.sync_copy(src.at[pl.ds(base, BPW)], src_v)
        pltpu.sync_copy(dst.at[pl.ds(base, BPW)], dst_v)
        for i in range(BPW // L):
            s16 = src_v[pl.ds(i * L, L)]
            d16 = dst_v[pl.ds(i * L, L)]
            idx_v[i // (CH // L), pl.ds((i % (CH // L)) * L, L)] = (
                s16 * n + lax.rem(d16, n))
        lane = lax.iota(jnp.int32, L)
        rvecs = [lane + k * L for k in range(CH // L)]
        cvecs = [jnp.full((L,), c, jnp.int32) for c in range(pw)]
        zero16 = jnp.zeros((L,), jnp.float32)

        def fire(j, buf):
            return [
                pltpu.async_copy(tabT.at[c].at[idx_v.at[j]],
                                 col_v.at[buf, c], sem)
                for c in range(pe)
            ]

        cps = fire(0, 0)
        for j in range(NCH):
            for cp in cps:
                cp.wait()
            if j + 1 < NCH:
                nxt = fire(j + 1, (j + 1) % 2)
            for c in range(pe):
                for k in range(CH // L):
                    x = col_v[j % 2, c, pl.ds(k * L, L)]
                    plsc.store_scatter(
                        rows_v, [rvecs[k] + j * CH, cvecs[c]], x)
            for c in range(pe, pw):
                for k in range(CH // L):
                    plsc.store_scatter(
                        rows_v, [rvecs[k] + j * CH, cvecs[c]], zero16)
            if j + 1 < NCH:
                cps = nxt
        zcp.wait()
        pltpu.sync_copy(zer_v, out.at[pl.ds(base, HR), pl.ds(pw, ZC)])
        pltpu.sync_copy(zer_v, out.at[pl.ds(base + HR, HR), pl.ds(pw, ZC)])
        pltpu.sync_copy(rows_v, out.at[pl.ds(base, BPW), pl.ds(0, pw)])

    return pl.kernel(
        body,
        mesh=mesh,
        out_type=jax.ShapeDtypeStruct((E, EMB_DIM), jnp.float32),
        scratch_types=[
            pltpu.VMEM((BPW,), jnp.int32),
            pltpu.VMEM((BPW,), jnp.int32),
            pltpu.VMEM((NCH, CH), jnp.int32),
            pltpu.VMEM((2, pe, CH), jnp.float32),
            pltpu.VMEM((BPW, pw), jnp.float32),
            pltpu.VMEM((HR, ZC), jnp.float32),
            pltpu.SemaphoreType.DMA,
            pltpu.SemaphoreType.DMA,
        ],
        compiler_params=pltpu.CompilerParams(
            use_tc_tiling_on_sc=False, needs_layout_passes=False),
    )


def kernel(edge_RWSE, batch, edge_index):
    total_nodes = batch.shape[0]
    n = edge_RWSE.shape[0] // total_nodes
    pe = edge_RWSE.shape[1]
    E = edge_index.shape[1]
    pw = ((pe + 7) // 8) * 8
    ei = edge_index.astype(jnp.int32)
    zsrc = jnp.zeros((E // 64, EMB_DIM - pw), jnp.float32)
    return _build(n, pe, pw, E)(edge_RWSE.T, ei[0], ei[1], zsrc)


# speedup vs baseline: 5.1540x; 1.1177x over previous
import functools

import jax
import jax.numpy as jnp
from jax import lax
from jax.experimental import pallas as pl
from jax.experimental.pallas import tpu as pltpu
from jax.experimental.pallas import tpu_sc as plsc

EMB_DIM = 128


@functools.lru_cache(maxsize=None)
def _build(n, pe, pw, E):
    info = plsc.get_sparse_core_info()
    NC, NS, L = info.num_cores, info.num_subcores, info.num_lanes
    NW = NC * NS
    BPW = E // NW
    CH = 128
    NCH = BPW // CH
    HR = BPW // 2
    ZC = EMB_DIM - pw

    mesh = plsc.VectorSubcoreMesh(core_axis_name="c", subcore_axis_name="s")

    D = 4
    ZR = 128

    def body(tabT, src, dst, zsrc, out, src_v, dst_v, idx_v, col_v, rows_v,
             zer_v, sems, zsem, wsem):
        wid = lax.axis_index("s") * NC + lax.axis_index("c")
        base = wid * BPW
        zcp = pltpu.async_copy(zsrc, zer_v, zsem)
        pltpu.sync_copy(src.at[pl.ds(base, BPW)], src_v)
        pltpu.sync_copy(dst.at[pl.ds(base, BPW)], dst_v)
        for i in range(BPW // L):
            s16 = src_v[pl.ds(i * L, L)]
            d16 = dst_v[pl.ds(i * L, L)]
            idx_v[i // (CH // L), pl.ds((i % (CH // L)) * L, L)] = (
                s16 * n + lax.rem(d16, n))
        lane = lax.iota(jnp.int32, L)
        rvecs = [lane + k * L for k in range(CH // L)]
        cvecs = [jnp.full((L,), c, jnp.int32) for c in range(pw)]
        zero16 = jnp.zeros((L,), jnp.float32)

        def fire(j):
            return [
                pltpu.async_copy(tabT.at[c].at[idx_v.at[j]],
                                 col_v.at[j % D, c], sems.at[j % D])
                for c in range(pe)
            ]

        ring = [fire(j) for j in range(D - 1)] + [None]
        zcp.wait()
        zcps = [
            pltpu.async_copy(
                zer_v, out.at[pl.ds(base + r * ZR, ZR), pl.ds(pw, ZC)],
                wsem)
            for r in range(BPW // ZR)
        ]
        for j in range(NCH):
            for cp in ring[j % D]:
                cp.wait()
            if j + D - 1 < NCH:
                ring[(j + D - 1) % D] = fire(j + D - 1)
            for c in range(pe):
                for k in range(CH // L):
                    x = col_v[j % D, c, pl.ds(k * L, L)]
                    plsc.store_scatter(
                        rows_v, [rvecs[k] + j * CH, cvecs[c]], x)
            for c in range(pe, pw):
                for k in range(CH // L):
                    plsc.store_scatter(
                        rows_v, [rvecs[k] + j * CH, cvecs[c]], zero16)
        pltpu.sync_copy(rows_v, out.at[pl.ds(base, BPW), pl.ds(0, pw)])
        for cp in zcps:
            cp.wait()

    return pl.kernel(
        body,
        mesh=mesh,
        out_type=jax.ShapeDtypeStruct((E, EMB_DIM), jnp.float32),
        scratch_types=[
            pltpu.VMEM((BPW,), jnp.int32),
            pltpu.VMEM((BPW,), jnp.int32),
            pltpu.VMEM((NCH, CH), jnp.int32),
            pltpu.VMEM((D, pe, CH), jnp.float32),
            pltpu.VMEM((BPW, pw), jnp.float32),
            pltpu.VMEM((ZR, ZC), jnp.float32),
            pltpu.SemaphoreType.DMA((D,)),
            pltpu.SemaphoreType.DMA,
            pltpu.SemaphoreType.DMA,
        ],
        compiler_params=pltpu.CompilerParams(
            use_tc_tiling_on_sc=False, needs_layout_passes=False),
    )


def kernel(edge_RWSE, batch, edge_index):
    total_nodes = batch.shape[0]
    n = edge_RWSE.shape[0] // total_nodes
    pe = edge_RWSE.shape[1]
    E = edge_index.shape[1]
    pw = ((pe + 7) // 8) * 8
    ei = edge_index.astype(jnp.int32)
    zsrc = jnp.zeros((128, EMB_DIM - pw), jnp.float32)
    return _build(n, pe, pw, E)(edge_RWSE.T, ei[0], ei[1], zsrc)
